# baseline (device time: 24766 ns/iter reference)
import os

import jax
import jax.numpy as jnp
from jax import lax
from jax.experimental import pallas as pl
from jax.experimental.pallas import tpu as pltpu

N_DEV = 32
GA = 8
GB = 4
F8 = jnp.float8_e4m3fn

VARIANT = os.environ.get("SCBAND_KVARIANT", "full")


def kernel(x, w_mat, scale_x, scale_w):
    m_tot, k_loc = x.shape
    k_tot, n = w_mat.shape
    m_per = m_tot // N_DEV
    k_blk = k_tot // GA

    def body(x_ref, w_ref, sx_ref, sw_ref, out_ref,
             w_stage, xs8_ref, p1_ref, comm2_ref,
             w_sems, p1_send_sems, p1_recv_sems,
             c2_send_sems, c2_recv_sems, ready_sems):
        my = lax.axis_index("i")
        my_a = my // GB
        my_b = lax.rem(my, GB)

        def issue_w_dma(j):
            bb = lax.rem(my_a + j, GA)
            dma = pltpu.make_async_copy(
                w_ref.at[pl.ds(bb * k_blk, k_blk), :],
                w_stage.at[pl.ds(bb * k_blk, k_blk), :],
                w_sems.at[j],
            )
            dma.start()
            return dma

        w_dmas = [issue_w_dma(j) for j in range(GA)]

        xs8_ref[...] = x_ref[...].reshape(GA, GB, m_per, k_loc).astype(F8)

        for db in range(1, GB):
            pl.semaphore_signal(
                ready_sems.at[my], inc=1,
                device_id=(my_a * GB + lax.rem(my_b + db, GB),),
                device_id_type=pl.DeviceIdType.MESH,
            )
        for da in range(1, GA):
            pl.semaphore_signal(
                ready_sems.at[my], inc=1,
                device_id=(lax.rem(my_a + da, GA) * GB + my_b,),
                device_id_type=pl.DeviceIdType.MESH,
            )

        p1_local = pltpu.make_async_copy(
            xs8_ref.at[:, my_b, :, :],
            p1_ref.at[my_b],
            p1_recv_sems.at[my_b],
        )
        p1_local.start()

        sends = []
        if VARIANT == "full":
            for db in range(1, GB):
                b2 = lax.rem(my_b + db, GB)
                dst = my_a * GB + b2
                pl.semaphore_wait(ready_sems.at[dst], 1)
                rdma = pltpu.make_async_remote_copy(
                    src_ref=xs8_ref.at[:, b2, :, :],
                    dst_ref=p1_ref.at[my_b],
                    send_sem=p1_send_sems.at[db],
                    recv_sem=p1_recv_sems.at[my_b],
                    device_id=(dst,),
                    device_id_type=pl.DeviceIdType.MESH,
                )
                rdma.start()
                sends.append(rdma)

            for bs in range(GB):
                recv = pltpu.make_async_remote_copy(
                    src_ref=xs8_ref.at[:, 0, :, :],
                    dst_ref=p1_ref.at[bs],
                    send_sem=p1_send_sems.at[0],
                    recv_sem=p1_recv_sems.at[bs],
                    device_id=(my,),
                    device_id_type=pl.DeviceIdType.MESH,
                )
                recv.wait_recv()

            c2_local = pltpu.make_async_copy(
                p1_ref.at[:, my_a, :, :],
                comm2_ref.at[my_a],
                c2_recv_sems.at[my_a],
            )
            c2_local.start()

            for da in range(1, GA):
                a2 = lax.rem(my_a + da, GA)
                dst = a2 * GB + my_b
                pl.semaphore_wait(ready_sems.at[dst], 1)
                rdma = pltpu.make_async_remote_copy(
                    src_ref=p1_ref.at[:, a2, :, :],
                    dst_ref=comm2_ref.at[my_a],
                    send_sem=c2_send_sems.at[da],
                    recv_sem=c2_recv_sems.at[my_a],
                    device_id=(dst,),
                    device_id_type=pl.DeviceIdType.MESH,
                )
                rdma.start()
                sends.append(rdma)

        acc = None
        for j in range(GA):
            bb = lax.rem(my_a + j, GA)
            w_dmas[j].wait()
            if VARIANT == "streamonly":
                continue

            recv = pltpu.make_async_remote_copy(
                src_ref=p1_ref.at[:, 0, :, :],
                dst_ref=comm2_ref.at[bb],
                send_sem=c2_send_sems.at[0],
                recv_sem=c2_recv_sems.at[bb],
                device_id=(my,),
                device_id_type=pl.DeviceIdType.MESH,
            )
            recv.wait_recv()

            xg_blk = jnp.concatenate(
                [comm2_ref[bb, bs] for bs in range(GB)],
                axis=1).astype(jnp.float32)
            term = jnp.dot(
                xg_blk,
                w_stage[pl.ds(bb * k_blk, k_blk), :],
                preferred_element_type=jnp.float32,
            )
            acc = term if acc is None else acc + term

        scale = sx_ref[0] * sw_ref[0]
        if acc is None:
            p1_local.wait()
            out_ref[...] = jnp.zeros((m_per, n), jnp.float32) + scale
        else:
            out_ref[...] = jnp.maximum(acc * scale, 0.0)

        for rdma in sends:
            rdma.wait_send()

    return pl.pallas_call(
        body,
        out_shape=jax.ShapeDtypeStruct((m_per, n), jnp.float32),
        in_specs=[
            pl.BlockSpec(memory_space=pltpu.VMEM),
            pl.BlockSpec(memory_space=pl.ANY),
            pl.BlockSpec(memory_space=pltpu.SMEM),
            pl.BlockSpec(memory_space=pltpu.SMEM),
        ],
        out_specs=pl.BlockSpec(memory_space=pltpu.VMEM),
        scratch_shapes=[
            pltpu.VMEM((k_tot, n), jnp.float32),
            pltpu.VMEM((GA, GB, m_per, k_loc), F8),
            pltpu.VMEM((GB, GA, m_per, k_loc), F8),
            pltpu.VMEM((GA, GB, m_per, k_loc), F8),
            pltpu.SemaphoreType.DMA((GA,)),
            pltpu.SemaphoreType.DMA((GB,)),
            pltpu.SemaphoreType.DMA((GB,)),
            pltpu.SemaphoreType.DMA((GA,)),
            pltpu.SemaphoreType.DMA((GA,)),
            pltpu.SemaphoreType.REGULAR((N_DEV,)),
        ],
        compiler_params=pltpu.CompilerParams(
            vmem_limit_bytes=64 * 1024 * 1024,
            skip_device_barrier=True,
        ),
    )(x, w_mat, scale_x, scale_w)
